# baseline (speedup 1.0000x reference)
import functools

import jax
import jax.numpy as jnp
import numpy as np
from jax import lax
from jax.experimental import pallas as pl
from jax.experimental.pallas import tpu as pltpu
from jax.experimental.pallas import tpu_sc as plsc

N = 10000
NP = 10240
K = 8
T = 512
DC = 12
DM = 16
NW = 32
SPT = NP // NW
GPT = SPT // 16
BS = 2048
SF = 0.2


def _prep_body(amps_ref, ph_ref, off_ref, tr_ref, tv_ref, tbl_ref, oft_ref,
               q_ref):
    ph = ph_ref[...]
    tbl_ref[...] = jnp.concatenate(
        [amps_ref[...], jnp.cos(ph), jnp.sin(ph)], axis=0)
    oft_ref[...] = jnp.concatenate([off_ref[...], tr_ref[...]], axis=0)
    coefs = [float(np.float32(2.0 * np.pi * f)) for f in (4.0, 2.0, 1.0, 0.5)]
    tv = tv_ref[...]
    args = jnp.concatenate([c * tv for c in coefs], axis=0)
    q_ref[...] = jnp.concatenate([jnp.sin(args), jnp.cos(args)], axis=0)


def _sc_mix_body(tbl_hbm, idx_hbm, w_hbm, oft_hbm, out_hbm,
                 tbl_l, idx_l, w_l, oft_l, out_l):
    wid = lax.axis_index("s") * 2 + lax.axis_index("c")
    pltpu.sync_copy(tbl_hbm, tbl_l)
    lane = lax.iota(jnp.int32, 16)
    HS = SPT // 2

    for h in range(2):
        sb = wid * SPT + h * HS
        pltpu.sync_copy(idx_hbm.at[pl.ds(sb * K, HS * K)], idx_l)
        pltpu.sync_copy(w_hbm.at[pl.ds(sb * K, HS * K)], w_l)
        pltpu.sync_copy(oft_hbm.at[pl.ds(sb * 2, HS * 2)], oft_l)

        def group(g, _):
            sl = g * 16 + lane
            acc = [jnp.zeros((16,), jnp.float32) for _ in range(DC)]
            for k in range(K):
                ik = sl * K + k
                nbr = plsc.load_gather(idx_l, [ik]) * DC
                wk = plsc.load_gather(w_l, [ik])
                for c in range(DC):
                    acc[c] = acc[c] + wk * plsc.load_gather(tbl_l, [nbr + c])
            own = jnp.minimum(sb + sl, N - 1) * DC
            ob = sl * DM
            for c in range(DC):
                mine = plsc.load_gather(tbl_l, [own + c])
                mixed = (1.0 - SF) * mine + SF * acc[c]
                plsc.store_scatter(out_l, [ob + c], mixed)
            for c in range(2):
                v = plsc.load_gather(oft_l, [sl * 2 + c])
                plsc.store_scatter(out_l, [ob + DC + c], v)
            zero = jnp.zeros((16,), jnp.float32)
            plsc.store_scatter(out_l, [ob + 14], zero)
            plsc.store_scatter(out_l, [ob + 15], zero)
            return 0

        lax.fori_loop(0, GPT // 2, group, 0)
        pltpu.sync_copy(out_l, out_hbm.at[pl.ds(sb * DM, HS * DM)])


def _main_body(m_ref, q_ref, tv_ref, out_ref):
    m = m_ref[...]
    amp_s = m[:, 0:4]
    re = m[:, 4:8]
    im = m[:, 8:12]
    h = jnp.sqrt(re * re + im * im)
    safe = h > 0.0
    inv = jnp.where(safe, 1.0 / jnp.where(safe, h, 1.0), 0.0)
    cph = jnp.where(safe, re * inv, 1.0)
    sph = im * inv
    ab = jnp.concatenate([amp_s * cph, amp_s * sph], axis=1)
    seasonal = jnp.dot(ab.astype(jnp.bfloat16), q_ref[...].astype(jnp.bfloat16),
                       preferred_element_type=jnp.float32)
    out_ref[...] = m[:, 12:13] + m[:, 13:14] * tv_ref[...] + seasonal


def _prep(amps_t, phases_t, off2, tr2, tv2):
    return pl.pallas_call(
        _prep_body,
        out_shape=[jax.ShapeDtypeStruct((DC, N), jnp.float32),
                   jax.ShapeDtypeStruct((2, N), jnp.float32),
                   jax.ShapeDtypeStruct((2 * 4, T), jnp.float32)],
    )(amps_t, phases_t, off2, tr2, tv2)


@functools.cache
def _mix_fn():
    return pl.kernel(
        _sc_mix_body,
        out_type=jax.ShapeDtypeStruct((NP * DM,), jnp.float32),
        mesh=plsc.VectorSubcoreMesh(core_axis_name="c", subcore_axis_name="s"),
        scratch_types=[
            pltpu.VMEM((N * DC,), jnp.float32),
            pltpu.VMEM((SPT // 2 * K,), jnp.int32),
            pltpu.VMEM((SPT // 2 * K,), jnp.float32),
            pltpu.VMEM((SPT,), jnp.float32),
            pltpu.VMEM((SPT // 2 * DM,), jnp.float32),
        ],
        compiler_params=pltpu.CompilerParams(
            use_tc_tiling_on_sc=False, needs_layout_passes=False),
    )


def _main(m128, q8, tv2):
    return pl.pallas_call(
        _main_body,
        grid=(pl.cdiv(N, BS),),
        in_specs=[
            pl.BlockSpec((BS, DM), lambda i: (i, 0)),
            pl.BlockSpec((2 * 4, T), lambda i: (0, 0)),
            pl.BlockSpec((1, T), lambda i: (0, 0)),
        ],
        out_specs=pl.BlockSpec((BS, T), lambda i: (i, 0)),
        out_shape=jax.ShapeDtypeStruct((N, T), jnp.float32),
    )(m128, q8, tv2)


def kernel(time_vector, constant_offset, linear_trend, seasonal_amplitudes,
           seasonal_phases, neighbor_weights, neighbor_indices):
    tv2 = time_vector.reshape(1, T)
    off2 = constant_offset.reshape(1, N)
    tr2 = linear_trend.reshape(1, N)
    tbl_t, oft_t, q8 = _prep(seasonal_amplitudes.T, seasonal_phases.T,
                             off2, tr2, tv2)
    tbl_lin = tbl_t.T.reshape(N * DC)
    oft_lin = jnp.pad(oft_t.T, ((0, NP - N), (0, 0))).reshape(NP * 2)
    idx_pad = jnp.pad(neighbor_indices.reshape(N * K), (0, (NP - N) * K))
    w_pad = jnp.pad(neighbor_weights.reshape(N * K), (0, (NP - N) * K))
    mixed = _mix_fn()(tbl_lin, idx_pad, w_pad, oft_lin)
    return _main(mixed.reshape(NP, DM), q8, tv2)
